# Initial kernel scaffold; baseline (speedup 1.0000x reference)
import jax
import jax.numpy as jnp
from jax.experimental import pallas as pl


def kernel(query, key, value, query_labels, value_labels, Wq, bq, Wk, bk, Wv, bv, Wm, bm):
    raise NotImplementedError("write your pallas kernel here")


# speedup vs baseline: 1.6374x; 1.6374x over previous
import jax
import jax.numpy as jnp
import numpy as np
from jax.experimental import pallas as pl

B = 1
N = 2048
D_MODEL = 1024
NUM_HEADS = 16
HEAD_DIM = D_MODEL // NUM_HEADS
QBLK = 256


def _qkv_kernel(xq_ref, xk_ref, xv_ref, wq_ref, wk_ref, wv_ref,
                bq_ref, bk_ref, bv_ref, q_ref, k_ref, v_ref):
    q_ref[...] = jnp.dot(xq_ref[...], wq_ref[...],
                         preferred_element_type=jnp.float32) + bq_ref[...]
    k_ref[...] = jnp.dot(xk_ref[...], wk_ref[...],
                         preferred_element_type=jnp.float32) + bk_ref[...]
    v_ref[...] = jnp.dot(xv_ref[...], wv_ref[...],
                         preferred_element_type=jnp.float32) + bv_ref[...]


def _attn_kernel(qlab_ref, vlab_ref, q_ref, k_ref, v_ref, o_ref):
    mask = qlab_ref[...] == vlab_ref[...]
    neg = jnp.where(mask, 0.0, -1e30)
    has = jnp.any(mask, axis=-1, keepdims=True).astype(jnp.float32)
    for h in range(NUM_HEADS):
        sl = slice(h * HEAD_DIM, (h + 1) * HEAD_DIM)
        q = q_ref[:, sl]
        k = k_ref[:, sl]
        s = jax.lax.dot_general(q, k, (((1,), (1,)), ((), ())),
                                preferred_element_type=jnp.float32) * 0.125
        masked = s + neg
        m = jnp.max(masked, axis=-1, keepdims=True)
        e = jnp.exp(masked - m)
        denom = jnp.sum(e, axis=-1, keepdims=True)
        p = e / denom
        o = jnp.dot(p, v_ref[:, sl], preferred_element_type=jnp.float32)
        o_ref[:, sl] = o * has


def _outproj_kernel(nv_ref, wm_ref, bm_ref, o_ref):
    o_ref[...] = jnp.dot(nv_ref[...], wm_ref[...],
                         preferred_element_type=jnp.float32) + bm_ref[...]


def _headmajor(W):
    return W.T.reshape(D_MODEL, HEAD_DIM, NUM_HEADS).transpose(0, 2, 1) \
              .reshape(D_MODEL, D_MODEL)


def _headmajor_b(b):
    return b.reshape(HEAD_DIM, NUM_HEADS).T.reshape(1, D_MODEL)


@jax.jit
def kernel(query, key, value, query_labels, value_labels,
           Wq, bq, Wk, bk, Wv, bv, Wm, bm):
    xq = query[0].T
    xk = key[0].T
    xv = value[0].T
    WqR, WkR, WvR = _headmajor(Wq), _headmajor(Wk), _headmajor(Wv)
    bqR, bkR, bvR = _headmajor_b(bq), _headmajor_b(bk), _headmajor_b(bv)
    WmRT = Wm.reshape(D_MODEL, HEAD_DIM, NUM_HEADS).transpose(0, 2, 1) \
             .reshape(D_MODEL, D_MODEL).T
    bmR = bm.reshape(1, D_MODEL)

    nblk = N // QBLK
    q2, k2, v2 = pl.pallas_call(
        _qkv_kernel,
        grid=(nblk,),
        in_specs=[
            pl.BlockSpec((QBLK, D_MODEL), lambda i: (i, 0)),
            pl.BlockSpec((QBLK, D_MODEL), lambda i: (i, 0)),
            pl.BlockSpec((QBLK, D_MODEL), lambda i: (i, 0)),
            pl.BlockSpec((D_MODEL, D_MODEL), lambda i: (0, 0)),
            pl.BlockSpec((D_MODEL, D_MODEL), lambda i: (0, 0)),
            pl.BlockSpec((D_MODEL, D_MODEL), lambda i: (0, 0)),
            pl.BlockSpec((1, D_MODEL), lambda i: (0, 0)),
            pl.BlockSpec((1, D_MODEL), lambda i: (0, 0)),
            pl.BlockSpec((1, D_MODEL), lambda i: (0, 0)),
        ],
        out_specs=[
            pl.BlockSpec((QBLK, D_MODEL), lambda i: (i, 0)),
            pl.BlockSpec((QBLK, D_MODEL), lambda i: (i, 0)),
            pl.BlockSpec((QBLK, D_MODEL), lambda i: (i, 0)),
        ],
        out_shape=[jax.ShapeDtypeStruct((N, D_MODEL), jnp.float32)] * 3,
    )(xq, xk, xv, WqR, WkR, WvR, bqR, bkR, bvR)

    qlab = query_labels[0].reshape(N, 1)
    vlab = value_labels[0].reshape(1, N)
    attn = pl.pallas_call(
        _attn_kernel,
        grid=(nblk,),
        in_specs=[
            pl.BlockSpec((QBLK, 1), lambda i: (i, 0)),
            pl.BlockSpec((1, N), lambda i: (0, 0)),
            pl.BlockSpec((QBLK, D_MODEL), lambda i: (i, 0)),
            pl.BlockSpec((N, D_MODEL), lambda i: (0, 0)),
            pl.BlockSpec((N, D_MODEL), lambda i: (0, 0)),
        ],
        out_specs=pl.BlockSpec((QBLK, D_MODEL), lambda i: (i, 0)),
        out_shape=jax.ShapeDtypeStruct((N, D_MODEL), jnp.float32),
    )(qlab, vlab, q2, k2, v2)

    outT = pl.pallas_call(
        _outproj_kernel,
        grid=(nblk,),
        in_specs=[
            pl.BlockSpec((QBLK, D_MODEL), lambda i: (i, 0)),
            pl.BlockSpec((D_MODEL, D_MODEL), lambda i: (0, 0)),
            pl.BlockSpec((1, D_MODEL), lambda i: (0, 0)),
        ],
        out_specs=pl.BlockSpec((QBLK, D_MODEL), lambda i: (i, 0)),
        out_shape=jax.ShapeDtypeStruct((N, D_MODEL), jnp.float32),
    )(attn, WmRT, bmR)

    return outT.T[None]
